# Initial kernel scaffold; baseline (speedup 1.0000x reference)
import jax
import jax.numpy as jnp
from jax.experimental import pallas as pl


def kernel(x, edge_indices, norms, d_idx, d_w, u_idx, u_w, params):
    raise NotImplementedError("write your pallas kernel here")


# speedup vs baseline: 5.1467x; 5.1467x over previous
import functools

import jax
import jax.numpy as jnp
from jax import lax
from jax.experimental import pallas as pl
from jax.experimental.pallas import tpu as pltpu
from jax.experimental.pallas import tpu_sc as plsc

NC, NS = 2, 16
CH = 128

_NPAD = {10000: 10240, 2500: 2560, 625: 768, 160: 256, 40: 256}
_WCH = {10240: 32, 2560: 80, 768: 48, 256: 16}
_CHP = {2560: 80, 768: 48, 256: 16}


def _prep_level(ei, n, n_pad):
    e = ei.shape[1]
    cap = -(-e // CH) * CH
    RPT = n_pad // NS
    src, dst = ei[0].astype(jnp.int32), ei[1].astype(jnp.int32)
    b = dst // RPT
    order = jnp.argsort(b, stable=True)
    ss, ds, bs = src[order], dst[order], b[order]
    start = jnp.searchsorted(bs, jnp.arange(NS, dtype=jnp.int32))
    cnt = jnp.diff(jnp.concatenate([start, jnp.array([e])])).astype(jnp.int32)
    rank = jnp.arange(e, dtype=jnp.int32) - start[bs].astype(jnp.int32)
    srcb = jnp.full((NS, cap), n, jnp.int32).at[bs, rank].set(ss)
    dstb = jnp.zeros((NS, cap), jnp.int32).at[bs, rank].set(ds - bs * RPT)
    nch = -(-cnt // CH)
    deg = jnp.zeros((n,), jnp.float32).at[ei[0]].add(1.0)
    dis = jnp.where(deg > 0, deg ** -0.5, 0.0)
    return srcb.reshape(-1), dstb.reshape(-1), nch, cap, dis


def _dis_col(dis, n_pad, NB):
    return jnp.tile(jnp.pad(dis, (0, n_pad - dis.shape[0])), NB).reshape(-1, 1)


@functools.lru_cache(None)
def _prop_kernel(n_pad, F, NB, E_cap):
    RPT = n_pad // NS
    WCH = _WCH[n_pad]
    n_w = RPT // WCH
    NBC = NB // NC
    FV = F // 16
    mesh = plsc.VectorSubcoreMesh(core_axis_name="c", subcore_axis_name="s",
                                  num_cores=NC, num_subcores=NS)

    @functools.partial(
        pl.kernel, mesh=mesh,
        out_type=jax.ShapeDtypeStruct((NB * n_pad * F,), jnp.float32),
        scratch_types=[
            pltpu.VMEM((RPT * F,), jnp.float32),
            pltpu.VMEM((CH,), jnp.int32),
            pltpu.VMEM((CH,), jnp.int32),
            pltpu.VMEM((CH, F), jnp.float32),
            pltpu.VMEM((CH + 16,), jnp.int32),
            pltpu.VMEM((NS + 16,), jnp.int32),
            pltpu.SemaphoreType.DMA,
        ],
    )
    def k(s_hbm, src_hbm, dst_hbm, cnt_hbm, out_hbm,
          acc, srcr, srcv, rows, dstv, cntv, sem):
        c = lax.axis_index("c")
        s = lax.axis_index("s")
        ebase = s * E_cap
        pltpu.sync_copy(cnt_hbm, cntv.at[pl.ds(0, NS)])
        nch = cntv[pl.ds(s, 16)][0]
        zvec = jnp.zeros((16,), jnp.float32)
        z16 = jnp.zeros((16,), jnp.int32)

        def batch_body(bi, _):
            slot = bi * NC + c
            base = slot * n_pad
            bvec = z16 + base

            def zacc(i, _):
                acc[pl.ds(i * 16, 16)] = zvec
                return 0
            lax.fori_loop(0, RPT * F // 16, zacc, 0)

            def chunk(ci, _):
                eo = ebase + ci * CH
                pltpu.sync_copy(src_hbm.at[pl.ds(eo, CH)], srcr)
                pltpu.sync_copy(dst_hbm.at[pl.ds(eo, CH)], dstv.at[pl.ds(0, CH)])

                def adj(j, _):
                    srcv[pl.ds(j * 16, 16)] = srcr[pl.ds(j * 16, 16)] + bvec
                    return 0
                lax.fori_loop(0, CH // 16, adj, 0)
                pltpu.async_copy(s_hbm.at[srcv], rows, sem).wait()

                def edge(e, _):
                    dF = dstv[pl.ds(e, 16)][0] * F
                    for j in range(FV):
                        o = dF + j * 16
                        acc[pl.ds(o, 16)] = (acc[pl.ds(o, 16)]
                                             + rows[e, pl.ds(j * 16, 16)])
                    return 0
                lax.fori_loop(0, CH, edge, 0)
                return 0
            lax.fori_loop(0, nch, chunk, 0)

            pltpu.sync_copy(
                acc, out_hbm.at[pl.ds((base + s * RPT) * F, RPT * F)])
            return 0
        lax.fori_loop(0, NBC, batch_body, 0)

    return k


@functools.lru_cache(None)
def _pool_gather_kernel(n_out_pad, F, NB):
    OPT = n_out_pad // NS
    CHP = _CHP[n_out_pad]
    n_pc = OPT // CHP
    NBC = NB // NC
    mesh = plsc.VectorSubcoreMesh(core_axis_name="c", subcore_axis_name="s",
                                  num_cores=NC, num_subcores=NS)

    @functools.partial(
        pl.kernel, mesh=mesh,
        out_type=jax.ShapeDtypeStruct((3, NB * n_out_pad, F), jnp.float32),
        scratch_types=[
            pltpu.VMEM((CHP,), jnp.int32),
            pltpu.VMEM((CHP, F), jnp.float32),
            pltpu.SemaphoreType.DMA,
        ],
    )
    def k(x_hbm, idx_hbm, out_hbm, idxv, grow, sem):
        c = lax.axis_index("c")
        s = lax.axis_index("s")

        def batch_body(bi, _):
            slot = bi * NC + c

            def chunk(ci, _):
                o0 = s * OPT + ci * CHP
                for j in range(3):
                    ioff = (slot * 3 + j) * n_out_pad + o0
                    pltpu.sync_copy(idx_hbm.at[pl.ds(ioff, CHP)], idxv)
                    pltpu.async_copy(x_hbm.at[idxv], grow, sem).wait()
                    pltpu.sync_copy(
                        grow, out_hbm.at[j, pl.ds(slot * n_out_pad + o0, CHP)])
                return 0
            lax.fori_loop(0, n_pc, chunk, 0)
            return 0
        lax.fori_loop(0, NBC, batch_body, 0)

    return k


def _pick_bm(M):
    for bm in (640, 512, 256, 128, 64, 32, 16, 8):
        if M % bm == 0:
            return bm
    return M


@functools.lru_cache(None)
def _scale_kernel(M, F):
    BM = _pick_bm(M)

    def body(a_ref, c_ref, o_ref):
        o_ref[...] = a_ref[...] * c_ref[...]

    return pl.pallas_call(
        body, grid=(M // BM,),
        in_specs=[pl.BlockSpec((BM, F), lambda m: (m, 0)),
                  pl.BlockSpec((BM, 1), lambda m: (m, 0))],
        out_specs=pl.BlockSpec((BM, F), lambda m: (m, 0)),
        out_shape=jax.ShapeDtypeStruct((M, F), jnp.float32))


@functools.lru_cache(None)
def _rec_kernel(M, F):
    BM = _pick_bm(M)

    def body(a_ref, p_ref, c_ref, o_ref):
        o_ref[...] = 2.0 * (c_ref[...] * a_ref[...]) - p_ref[...]

    return pl.pallas_call(
        body, grid=(M // BM,),
        in_specs=[pl.BlockSpec((BM, F), lambda m: (m, 0)),
                  pl.BlockSpec((BM, F), lambda m: (m, 0)),
                  pl.BlockSpec((BM, 1), lambda m: (m, 0))],
        out_specs=pl.BlockSpec((BM, F), lambda m: (m, 0)),
        out_shape=jax.ShapeDtypeStruct((M, F), jnp.float32))


@functools.lru_cache(None)
def _combine_kernel(M, F):
    BM = _pick_bm(M)

    def body(g_ref, w0_ref, w1_ref, w2_ref, d_ref, t_ref, s_ref):
        g = g_ref[...]
        t = g[0] * w0_ref[...] + g[1] * w1_ref[...] + g[2] * w2_ref[...]
        t_ref[...] = t
        s_ref[...] = d_ref[...] * t

    wspec = pl.BlockSpec((BM, 1), lambda m: (m, 0))
    fspec = pl.BlockSpec((BM, F), lambda m: (m, 0))
    return pl.pallas_call(
        body, grid=(M // BM,),
        in_specs=[pl.BlockSpec((3, BM, F), lambda m: (0, m, 0)),
                  wspec, wspec, wspec, wspec],
        out_specs=(fspec, fspec),
        out_shape=(jax.ShapeDtypeStruct((M, F), jnp.float32),
                   jax.ShapeDtypeStruct((M, F), jnp.float32)))


@functools.lru_cache(None)
def _mm7_kernel(M, F, F2, relu, has_bias):
    BM = _pick_bm(M)

    def body(x_ref, a0, a1, a2, a3, a4, d_ref, w_ref, b_ref, out_ref):
        d = d_ref[...]
        acc = jnp.dot(x_ref[...], w_ref[0], preferred_element_type=jnp.float32)
        for j, ar in enumerate((a0, a1, a2, a3, a4)):
            acc = acc + jnp.dot(d * ar[...], w_ref[j + 1],
                                preferred_element_type=jnp.float32)
        if has_bias:
            acc = acc + b_ref[...]
        if relu:
            acc = jnp.maximum(acc, 0.0)
        out_ref[...] = acc

    fspec = pl.BlockSpec((BM, F), lambda m: (m, 0))
    return pl.pallas_call(
        body, grid=(M // BM,),
        in_specs=[fspec] * 6 + [
            pl.BlockSpec((BM, 1), lambda m: (m, 0)),
            pl.BlockSpec((6, F, F2), lambda m: (0, 0, 0)),
            pl.BlockSpec((1, F2), lambda m: (0, 0)),
        ],
        out_specs=pl.BlockSpec((BM, F2), lambda m: (m, 0)),
        out_shape=jax.ShapeDtypeStruct((M, F2), jnp.float32))


@functools.lru_cache(None)
def _mm_kernel(M, Kd, N, relu):
    def body(x_ref, w_ref, b_ref, out_ref):
        acc = jnp.dot(x_ref[...], w_ref[...], preferred_element_type=jnp.float32)
        acc = acc + b_ref[...]
        if relu:
            acc = jnp.maximum(acc, 0.0)
        out_ref[...] = acc

    return pl.pallas_call(
        body, out_shape=jax.ShapeDtypeStruct((M, N), jnp.float32))


def _comb_w(W):
    W0, W1, W2, W3, W4, W5 = [W[i] for i in range(6)]
    return jnp.stack([W0 - W2 + W4, W1 - W3 + W5, 2.0 * (W2 - W4),
                      2.0 * (W3 - W5), 2.0 * W4, 2.0 * W5])


def _cheb(t0, s0, srcb, dstb, nch, cap, n_pad, F, NB, W, b, relu, dcol, d2col):
    M = NB * n_pad
    pk = _prop_kernel(n_pad, F, NB, cap)
    prop = lambda sx: pk(sx, srcb, dstb, nch).reshape(M, F)
    a = [prop(s0)]
    s_prev2, s_prev = s0, _scale_kernel(M, F)(a[0], d2col)
    for _ in range(3):
        a.append(prop(s_prev))
        s_new = _rec_kernel(M, F)(a[-1], s_prev2, d2col)
        s_prev2, s_prev = s_prev, s_new
    a.append(prop(s_prev))
    F2 = W.shape[2]
    has_bias = b is not None
    bias = b.reshape(1, F2) if has_bias else jnp.zeros((1, F2), jnp.float32)
    return _mm7_kernel(M, F, F2, relu, has_bias)(
        t0, *a, dcol, _comb_w(W), bias)


def _pool(cur_t, idx, w, n_in_pad, n_out_pad, F, NB, dcol):
    n_out = idx.shape[0]
    idxp = jnp.pad(idx.astype(jnp.int32), ((0, n_out_pad - n_out), (0, 0)))
    idxg = (idxp.T[None, :, :]
            + (jnp.arange(NB, dtype=jnp.int32) * n_in_pad)[:, None, None]
            ).reshape(-1)
    wp = jnp.pad(w, ((0, n_out_pad - n_out), (0, 0)))
    wcols = [jnp.tile(wp[:, j], NB).reshape(-1, 1) for j in range(3)]
    g = _pool_gather_kernel(n_out_pad, F, NB)(cur_t, idxg)
    return _combine_kernel(NB * n_out_pad, F)(g, *wcols, dcol)


def kernel(x, edge_indices, norms, d_idx, d_w, u_idx, u_w, params):
    B_, T_, N0, F0 = x.shape
    NB = B_ * T_
    sizes = [10000, 2500, 625, 160, 40]
    filts = [128, 128, 128, 256, 256]
    Z = params['W_mu'].shape[1]

    npads = [_NPAD[s] for s in sizes]
    lev = [_prep_level(edge_indices[i], sizes[i], npads[i]) for i in range(4)]

    n0p = npads[0]
    cur_t = jnp.pad(x.reshape(NB, N0, F0),
                    ((0, 0), (0, n0p - N0), (0, 0))).reshape(NB * n0p, F0)
    dcols = [_dis_col(lev[i][4], npads[i], NB) for i in range(4)]
    cur_s = _scale_kernel(NB * n0p, F0)(cur_t, dcols[0])
    ones_bot = jnp.ones((NB * npads[4], 1), jnp.float32)
    for i in range(4):
        srcb, dstb, nch, cap, _ = lev[i]
        d2 = dcols[i] * dcols[i]
        out = _cheb(cur_t, cur_s, srcb, dstb, nch, cap, npads[i], filts[i], NB,
                    params['W_enc%d' % i], params['b_enc%d' % i], True,
                    dcols[i], d2)
        nxt_dcol = dcols[i + 1] if i < 3 else ones_bot
        cur_t, cur_s = _pool(out, d_idx[i], d_w[i], npads[i], npads[i + 1],
                             filts[i + 1], NB, nxt_dcol)

    nbp = npads[4]
    xb = cur_t.reshape(NB, nbp, filts[4])[:, :sizes[4], :].reshape(NB, -1)
    X2 = xb.reshape(B_, T_, -1).transpose(1, 0, 2).reshape(NB, -1)
    mu = _mm_kernel(NB, X2.shape[1], Z, False)(
        X2, params['W_mu'], params['b_mu'].reshape(1, Z))
    z = jnp.mean(mu.reshape(B_, T_, Z), axis=1)

    zp = jnp.pad(z, ((0, 8 - B_), (0, 0)))
    nfz = params['W_declin'].shape[1]
    y0 = _mm_kernel(8, Z, nfz, True)(
        zp, params['W_declin'], params['b_declin'].reshape(1, nfz))[:B_]
    y = y0.reshape(B_, sizes[4], filts[4])
    y = jnp.pad(y, ((0, 0), (0, nbp - sizes[4]), (0, 0))
                ).reshape(B_ * nbp, filts[4])

    dec_filts = [(256, 256), (256, 128), (128, 128)]
    ddcols = {lvl: _dis_col(lev[lvl][4], npads[lvl], B_) for lvl in (1, 2, 3)}
    for i in range(3):
        lvl = 3 - i
        fin, _ = dec_filts[i]
        cur_t, cur_s = _pool(y, u_idx[lvl], u_w[lvl],
                             npads[lvl + 1] if lvl < 3 else npads[4],
                             npads[lvl], fin, B_, ddcols[lvl])
        srcb, dstb, nch, cap, _ = lev[lvl]
        d2 = ddcols[lvl] * ddcols[lvl]
        y = _cheb(cur_t, cur_s, srcb, dstb, nch, cap, npads[lvl], fin, B_,
                  params['W_dec%d' % i], params['b_dec%d' % i], True,
                  ddcols[lvl], d2)

    srcb, dstb, nch, cap, disq = _prep_level(edge_indices[3], sizes[3],
                                             npads[1])
    dcolq = _dis_col(disq, npads[1], B_)
    yq_s = _scale_kernel(B_ * npads[1], 128)(y, dcolq)
    y = _cheb(y, yq_s, srcb, dstb, nch, cap, npads[1], 128, B_,
              params['W_dec3'], None, False, dcolq, dcolq * dcolq)
    y = y.reshape(B_, npads[1], 128)[:, :sizes[1], :].reshape(-1, 128)
    return z, y
